# Initial kernel scaffold; baseline (speedup 1.0000x reference)
import jax
import jax.numpy as jnp
from jax.experimental import pallas as pl


def kernel(x, edge_index, W_l, b_l, W_r, W_mlp, b_mlp):
    raise NotImplementedError("write your pallas kernel here")


# speedup vs baseline: 3.5637x; 3.5637x over previous
import functools

import jax
import jax.numpy as jnp
from jax import lax
from jax.experimental import pallas as pl
from jax.experimental.pallas import tpu as pltpu
from jax.experimental.pallas import tpu_sc as plsc

N = 10000
E = 320000
D = 128
H = 128
C = 64
HD = D // 2

NUM_CORES = 2
NUM_SUBCORES = 16
CHUNK = 80
CNT_W = 16
N_PAD = 10240
ROWS_PER_TILE = N_PAD // NUM_SUBCORES
TILE_ROUNDS = ROWS_PER_TILE // CHUNK

FEAT_EDGES_PER_TILE = E // NUM_SUBCORES
FEAT_CHUNKS = FEAT_EDGES_PER_TILE // CHUNK
CNT_EDGES_PER_TILE = E // (NUM_CORES * NUM_SUBCORES)
CNT_CHUNKS = CNT_EDGES_PER_TILE // CHUNK

_SC_MESH = dict(core_axis_name="c", subcore_axis_name="s")


def _sc_feature_sums(xcat, src2, dst, zrows):
  @functools.partial(
      pl.kernel,
      mesh=plsc.VectorSubcoreMesh(**_SC_MESH),
      compiler_params=pltpu.CompilerParams(use_tc_tiling_on_sc=False),
      out_type=jax.ShapeDtypeStruct((NUM_CORES, N_PAD, HD), jnp.float32),
      scratch_types=[
          pltpu.VMEM((CHUNK,), jnp.int32),
          pltpu.VMEM((CHUNK,), jnp.int32),
          pltpu.VMEM((CHUNK, HD), jnp.float32),
          pltpu.VMEM_SHARED((N_PAD, HD), jnp.float32),
          pltpu.SemaphoreType.DMA,
      ],
  )
  def feat_kernel(xcat_hbm, src2_hbm, dst_hbm, zrows_hbm, agg_hbm,
                  sidx, didx, rows, acc_sh, sem):
    cid = lax.axis_index("c")
    sid = lax.axis_index("s")
    row0 = sid * ROWS_PER_TILE

    pltpu.sync_copy(zrows_hbm, rows)
    for k in range(TILE_ROUNDS):
      pltpu.sync_copy(rows, acc_sh.at[pl.ds(row0 + k * CHUNK, CHUNK)])
    plsc.subcore_barrier()

    ebase = cid * E + sid * FEAT_EDGES_PER_TILE

    def body(j, carry):
      base = j * CHUNK
      pltpu.sync_copy(src2_hbm.at[pl.ds(ebase + base, CHUNK)], sidx)
      pltpu.sync_copy(dst_hbm.at[pl.ds(sid * FEAT_EDGES_PER_TILE + base, CHUNK)],
                      didx)
      pltpu.async_copy(xcat_hbm.at[sidx], rows, sem).wait()
      pltpu.sync_copy(rows, acc_sh.at[didx], add=True)
      return carry

    lax.fori_loop(0, FEAT_CHUNKS, body, 0)
    plsc.subcore_barrier()

    for k in range(TILE_ROUNDS):
      pltpu.sync_copy(acc_sh.at[pl.ds(row0 + k * CHUNK, CHUNK)], rows)
      pltpu.sync_copy(rows, agg_hbm.at[cid, pl.ds(row0 + k * CHUNK, CHUNK)])

  return feat_kernel(xcat, src2, dst, zrows)


def _sc_counts(dst, zcnt, ones_chunk):
  @functools.partial(
      pl.kernel,
      mesh=plsc.VectorSubcoreMesh(**_SC_MESH),
      compiler_params=pltpu.CompilerParams(use_tc_tiling_on_sc=False),
      out_type=jax.ShapeDtypeStruct((NUM_CORES, N_PAD, CNT_W), jnp.float32),
      scratch_types=[
          pltpu.VMEM((CHUNK,), jnp.int32),
          pltpu.VMEM((CHUNK, CNT_W), jnp.float32),
          pltpu.VMEM_SHARED((N_PAD, CNT_W), jnp.float32),
      ],
  )
  def cnt_kernel(dst_hbm, zcnt_hbm, ones_hbm, cntp_hbm, didx, ones_v, cnt_sh):
    cid = lax.axis_index("c")
    sid = lax.axis_index("s")
    row0 = sid * ROWS_PER_TILE

    pltpu.sync_copy(zcnt_hbm, ones_v)
    for k in range(TILE_ROUNDS):
      pltpu.sync_copy(ones_v, cnt_sh.at[pl.ds(row0 + k * CHUNK, CHUNK)])
    pltpu.sync_copy(ones_hbm, ones_v)
    plsc.subcore_barrier()

    ebase = (cid * NUM_SUBCORES + sid) * CNT_EDGES_PER_TILE

    def body(j, carry):
      pltpu.sync_copy(dst_hbm.at[pl.ds(ebase + j * CHUNK, CHUNK)], didx)
      pltpu.sync_copy(ones_v, cnt_sh.at[didx], add=True)
      return carry

    lax.fori_loop(0, CNT_CHUNKS, body, 0)
    plsc.subcore_barrier()

    for k in range(TILE_ROUNDS):
      pltpu.sync_copy(cnt_sh.at[pl.ds(row0 + k * CHUNK, CHUNK)], ones_v)
      pltpu.sync_copy(ones_v, cntp_hbm.at[cid, pl.ds(row0 + k * CHUNK, CHUNK)])

  return cnt_kernel(dst, zcnt, ones_chunk)


ROW_BLK = 1000


def _head_kernel(x_ref, agg_ref, cntp_ref, wl_ref, bl_ref, wr_ref,
                 wmlp_ref, bmlp_ref, out_ref):
  cnt = cntp_ref[0, :, 0] + cntp_ref[1, :, 0]
  recip = 1.0 / jnp.maximum(cnt, 1.0)
  mean0 = agg_ref[0] * recip[:, None]
  mean1 = agg_ref[1] * recip[:, None]
  wl = wl_ref[...]
  dn = (((1,), (1,)), ((), ()))
  h = lax.dot_general(mean0, wl[:, :HD], dn, preferred_element_type=jnp.float32)
  h += lax.dot_general(mean1, wl[:, HD:], dn, preferred_element_type=jnp.float32)
  h += lax.dot_general(x_ref[...], wr_ref[...], dn,
                       preferred_element_type=jnp.float32)
  h = jnp.maximum(h + bl_ref[...], 0.0)
  out_ref[...] = lax.dot_general(
      h, wmlp_ref[...], dn, preferred_element_type=jnp.float32) + bmlp_ref[...]


def _tc_head(x, agg, cntp, W_l, b_l, W_r, W_mlp, b_mlp):
  grid = (N // ROW_BLK,)
  return pl.pallas_call(
      _head_kernel,
      grid=grid,
      in_specs=[
          pl.BlockSpec((ROW_BLK, D), lambda i: (i, 0)),
          pl.BlockSpec((NUM_CORES, ROW_BLK, HD), lambda i: (0, i, 0)),
          pl.BlockSpec((NUM_CORES, ROW_BLK, CNT_W), lambda i: (0, i, 0)),
          pl.BlockSpec((H, D), lambda i: (0, 0)),
          pl.BlockSpec((1, H), lambda i: (0, 0)),
          pl.BlockSpec((H, D), lambda i: (0, 0)),
          pl.BlockSpec((C, H), lambda i: (0, 0)),
          pl.BlockSpec((1, C), lambda i: (0, 0)),
      ],
      out_specs=pl.BlockSpec((ROW_BLK, C), lambda i: (i, 0)),
      out_shape=jax.ShapeDtypeStruct((N, C), jnp.float32),
  )(x, agg, cntp, W_l, b_l.reshape(1, H), W_r, W_mlp, b_mlp.reshape(1, C))


@jax.jit
def kernel(x, edge_index, W_l, b_l, W_r, W_mlp, b_mlp):
  src = edge_index[0]
  dst = edge_index[1]
  xcat = jnp.concatenate([x[:, :HD], x[:, HD:]], axis=0)
  src2 = jnp.concatenate([src, src + N])
  zrows = jnp.zeros((CHUNK, HD), jnp.float32)
  zcnt = jnp.zeros((CHUNK, CNT_W), jnp.float32)
  ones_chunk = jnp.ones((CHUNK, CNT_W), jnp.float32)
  agg = _sc_feature_sums(xcat, src2, dst, zrows)
  cntp = _sc_counts(dst, zcnt, ones_chunk)
  return _tc_head(x, agg, cntp, W_l, b_l, W_r, W_mlp, b_mlp)


# speedup vs baseline: 10.2005x; 2.8624x over previous
import functools

import jax
import jax.numpy as jnp
from jax import lax
from jax.experimental import pallas as pl
from jax.experimental.pallas import tpu as pltpu
from jax.experimental.pallas import tpu_sc as plsc

N = 10000
E = 320000
D = 128
H = 128
C = 64
HD = D // 2

NUM_CORES = 2
NUM_SUBCORES = 16
CHUNK = 80
CNT_W = 16
N_PAD = 10240
ROWS_PER_TILE = N_PAD // NUM_SUBCORES
TILE_ROUNDS = ROWS_PER_TILE // CHUNK

FEAT_EDGES_PER_TILE = E // NUM_SUBCORES
FEAT_CHUNKS = FEAT_EDGES_PER_TILE // CHUNK
CNT_EDGES_PER_TILE = E // (NUM_CORES * NUM_SUBCORES)
CNT_CHUNKS = CNT_EDGES_PER_TILE // CHUNK

_SC_MESH = dict(core_axis_name="c", subcore_axis_name="s")
NBUF = 5


def _sc_feature_sums(xcat, src2r, dstr, zrows):
  @functools.partial(
      pl.kernel,
      mesh=plsc.VectorSubcoreMesh(**_SC_MESH),
      compiler_params=pltpu.CompilerParams(use_tc_tiling_on_sc=False),
      out_type=jax.ShapeDtypeStruct((NUM_CORES, N_PAD, HD), jnp.float32),
      scratch_types=[
          pltpu.VMEM((FEAT_CHUNKS, CHUNK), jnp.int32),
          pltpu.VMEM((FEAT_CHUNKS, CHUNK), jnp.int32),
          pltpu.VMEM((NBUF, CHUNK, HD), jnp.float32),
          pltpu.VMEM_SHARED((N_PAD, HD), jnp.float32),
      ] + [pltpu.SemaphoreType.DMA] * (2 * NBUF),
  )
  def feat_kernel(xcat_hbm, src2r_hbm, dstr_hbm, zrows_hbm, agg_hbm,
                  sidx, didx, rows, acc_sh, *sems):
    gsem = sems[:NBUF]
    ssem = sems[NBUF:]
    cid = lax.axis_index("c")
    sid = lax.axis_index("s")
    row0 = sid * ROWS_PER_TILE

    pltpu.sync_copy(src2r_hbm.at[cid, sid], sidx)
    pltpu.sync_copy(dstr_hbm.at[sid], didx)

    pltpu.sync_copy(zrows_hbm, rows.at[0])
    for k in range(TILE_ROUNDS):
      pltpu.sync_copy(rows.at[0], acc_sh.at[pl.ds(row0 + k * CHUNK, CHUNK)])
    plsc.subcore_barrier()

    def gather(j, b):
      pltpu.async_copy(xcat_hbm.at[sidx.at[j]], rows.at[b], gsem[b])

    def wait_gather(b):
      pltpu.make_async_copy(xcat_hbm.at[sidx.at[0]], rows.at[b], gsem[b]).wait()

    def wait_scatter(b):
      pltpu.make_async_copy(zrows_hbm, rows.at[b], ssem[b]).wait()

    for b in range(NBUF):
      gather(b, b)

    def body(t, carry):
      j0 = NBUF * t
      for b in range(NBUF):
        wait_gather(b)
        pltpu.async_copy(rows.at[b], acc_sh.at[didx.at[j0 + b]], ssem[b],
                         add=True)
      for b in range(NBUF):
        wait_scatter(b)
        nxt = j0 + NBUF + b
        gather(jnp.minimum(nxt, FEAT_CHUNKS - 1), b)
      return carry

    lax.fori_loop(0, FEAT_CHUNKS // NBUF, body, 0)
    for b in range(NBUF):
      wait_gather(b)
    plsc.subcore_barrier()

    for k in range(TILE_ROUNDS):
      pltpu.sync_copy(acc_sh.at[pl.ds(row0 + k * CHUNK, CHUNK)], rows.at[0])
      pltpu.sync_copy(rows.at[0], agg_hbm.at[cid, pl.ds(row0 + k * CHUNK, CHUNK)])

  return feat_kernel(xcat, src2r, dstr, zrows)


def _sc_counts(dst2r, zcnt, ones_chunk):
  @functools.partial(
      pl.kernel,
      mesh=plsc.VectorSubcoreMesh(**_SC_MESH),
      compiler_params=pltpu.CompilerParams(use_tc_tiling_on_sc=False),
      out_type=jax.ShapeDtypeStruct((NUM_CORES, N_PAD, CNT_W), jnp.float32),
      scratch_types=[
          pltpu.VMEM((CNT_CHUNKS, CHUNK), jnp.int32),
          pltpu.VMEM((CHUNK, CNT_W), jnp.float32),
          pltpu.VMEM_SHARED((N_PAD, CNT_W), jnp.float32),
          pltpu.SemaphoreType.DMA,
      ],
  )
  def cnt_kernel(dst2r_hbm, zcnt_hbm, ones_hbm, cntp_hbm, didx, ones_v,
                 cnt_sh, sem):
    cid = lax.axis_index("c")
    sid = lax.axis_index("s")
    row0 = sid * ROWS_PER_TILE

    pltpu.sync_copy(dst2r_hbm.at[cid, sid], didx)
    pltpu.sync_copy(zcnt_hbm, ones_v)
    for k in range(TILE_ROUNDS):
      pltpu.sync_copy(ones_v, cnt_sh.at[pl.ds(row0 + k * CHUNK, CHUNK)])
    pltpu.sync_copy(ones_hbm, ones_v)
    plsc.subcore_barrier()

    def body(t, carry):
      j0 = NBUF * t
      for b in range(NBUF):
        pltpu.async_copy(ones_v, cnt_sh.at[didx.at[j0 + b]], sem, add=True)
      for b in range(NBUF):
        pltpu.make_async_copy(zcnt_hbm, ones_v, sem).wait()
      return carry

    lax.fori_loop(0, CNT_CHUNKS // NBUF, body, 0)
    plsc.subcore_barrier()

    for k in range(TILE_ROUNDS):
      pltpu.sync_copy(cnt_sh.at[pl.ds(row0 + k * CHUNK, CHUNK)], ones_v)
      pltpu.sync_copy(ones_v, cntp_hbm.at[cid, pl.ds(row0 + k * CHUNK, CHUNK)])

  return cnt_kernel(dst2r, zcnt, ones_chunk)


ROW_BLK = 1000


def _head_kernel(x_ref, agg_ref, cntp_ref, wl_ref, bl_ref, wr_ref,
                 wmlp_ref, bmlp_ref, out_ref):
  cnt = cntp_ref[0, :, 0] + cntp_ref[1, :, 0]
  recip = 1.0 / jnp.maximum(cnt, 1.0)
  mean0 = agg_ref[0] * recip[:, None]
  mean1 = agg_ref[1] * recip[:, None]
  wl = wl_ref[...]
  dn = (((1,), (1,)), ((), ()))
  h = lax.dot_general(mean0, wl[:, :HD], dn, preferred_element_type=jnp.float32)
  h += lax.dot_general(mean1, wl[:, HD:], dn, preferred_element_type=jnp.float32)
  h += lax.dot_general(x_ref[...], wr_ref[...], dn,
                       preferred_element_type=jnp.float32)
  h = jnp.maximum(h + bl_ref[...], 0.0)
  out_ref[...] = lax.dot_general(
      h, wmlp_ref[...], dn, preferred_element_type=jnp.float32) + bmlp_ref[...]


def _tc_head(x, agg, cntp, W_l, b_l, W_r, W_mlp, b_mlp):
  grid = (N // ROW_BLK,)
  return pl.pallas_call(
      _head_kernel,
      grid=grid,
      in_specs=[
          pl.BlockSpec((ROW_BLK, D), lambda i: (i, 0)),
          pl.BlockSpec((NUM_CORES, ROW_BLK, HD), lambda i: (0, i, 0)),
          pl.BlockSpec((NUM_CORES, ROW_BLK, CNT_W), lambda i: (0, i, 0)),
          pl.BlockSpec((H, D), lambda i: (0, 0)),
          pl.BlockSpec((1, H), lambda i: (0, 0)),
          pl.BlockSpec((H, D), lambda i: (0, 0)),
          pl.BlockSpec((C, H), lambda i: (0, 0)),
          pl.BlockSpec((1, C), lambda i: (0, 0)),
      ],
      out_specs=pl.BlockSpec((ROW_BLK, C), lambda i: (i, 0)),
      out_shape=jax.ShapeDtypeStruct((N, C), jnp.float32),
  )(x, agg, cntp, W_l, b_l.reshape(1, H), W_r, W_mlp, b_mlp.reshape(1, C))


@jax.jit
def kernel(x, edge_index, W_l, b_l, W_r, W_mlp, b_mlp):
  src = edge_index[0]
  dst = edge_index[1]
  xcat = jnp.concatenate([x[:, :HD], x[:, HD:]], axis=0)
  src2r = jnp.concatenate([src, src + N]).reshape(
      NUM_CORES, NUM_SUBCORES, FEAT_CHUNKS, CHUNK)
  dstr = dst.reshape(NUM_SUBCORES, FEAT_CHUNKS, CHUNK)
  dst2r = dst.reshape(NUM_CORES, NUM_SUBCORES, CNT_CHUNKS, CHUNK)
  zrows = jnp.zeros((CHUNK, HD), jnp.float32)
  zcnt = jnp.zeros((CHUNK, CNT_W), jnp.float32)
  ones_chunk = jnp.ones((CHUNK, CNT_W), jnp.float32)
  agg = _sc_feature_sums(xcat, src2r, dstr, zrows)
  cntp = _sc_counts(dst2r, zcnt, ones_chunk)
  return _tc_head(x, agg, cntp, W_l, b_l, W_r, W_mlp, b_mlp)


# speedup vs baseline: 10.2915x; 1.0089x over previous
import functools

import jax
import jax.numpy as jnp
from jax import lax
from jax.experimental import pallas as pl
from jax.experimental.pallas import tpu as pltpu
from jax.experimental.pallas import tpu_sc as plsc

N = 10000
E = 320000
D = 128
H = 128
C = 64
HD = D // 2

NUM_CORES = 2
NUM_SUBCORES = 16
CHUNK = 80
CNT_W = 16
N_PAD = 10240
ROWS_PER_TILE = N_PAD // NUM_SUBCORES
TILE_ROUNDS = ROWS_PER_TILE // CHUNK

FEAT_EDGES_PER_TILE = E // NUM_SUBCORES
FEAT_CHUNKS = FEAT_EDGES_PER_TILE // CHUNK
CNT_EDGES_PER_TILE = E // (NUM_CORES * NUM_SUBCORES)
CNT_CHUNKS = CNT_EDGES_PER_TILE // CHUNK

_SC_MESH = dict(core_axis_name="c", subcore_axis_name="s")
NBUF = 5
CNT_BATCH = 5


def _sc_feature_sums(xcat, src2r, dstr, zrows):
  @functools.partial(
      pl.kernel,
      mesh=plsc.VectorSubcoreMesh(**_SC_MESH),
      compiler_params=pltpu.CompilerParams(use_tc_tiling_on_sc=False),
      out_type=jax.ShapeDtypeStruct((NUM_CORES, N_PAD, HD), jnp.float32),
      scratch_types=[
          pltpu.VMEM((FEAT_CHUNKS, CHUNK), jnp.int32),
          pltpu.VMEM((FEAT_CHUNKS, CHUNK), jnp.int32),
          pltpu.VMEM((NBUF, CHUNK, HD), jnp.float32),
          pltpu.VMEM_SHARED((N_PAD, HD), jnp.float32),
      ] + [pltpu.SemaphoreType.DMA] * (2 * NBUF),
  )
  def feat_kernel(xcat_hbm, src2r_hbm, dstr_hbm, zrows_hbm, agg_hbm,
                  sidx, didx, rows, acc_sh, *sems):
    gsem = sems[:NBUF]
    ssem = sems[NBUF:]
    cid = lax.axis_index("c")
    sid = lax.axis_index("s")
    row0 = sid * ROWS_PER_TILE

    pltpu.sync_copy(src2r_hbm.at[cid, sid], sidx)
    pltpu.sync_copy(dstr_hbm.at[sid], didx)

    pltpu.sync_copy(zrows_hbm, rows.at[0])
    for k in range(TILE_ROUNDS):
      pltpu.async_copy(rows.at[0], acc_sh.at[pl.ds(row0 + k * CHUNK, CHUNK)],
                       gsem[0])
    for k in range(TILE_ROUNDS):
      pltpu.make_async_copy(zrows_hbm, rows.at[0], gsem[0]).wait()
    plsc.subcore_barrier()

    def gather(j, b):
      pltpu.async_copy(xcat_hbm.at[sidx.at[j]], rows.at[b], gsem[b])

    def wait_gather(b):
      pltpu.make_async_copy(xcat_hbm.at[sidx.at[0]], rows.at[b], gsem[b]).wait()

    def wait_scatter(b):
      pltpu.make_async_copy(zrows_hbm, rows.at[b], ssem[b]).wait()

    for b in range(NBUF):
      gather(b, b)

    def body(t, carry):
      j0 = NBUF * t
      for b in range(NBUF):
        wait_gather(b)
        pltpu.async_copy(rows.at[b], acc_sh.at[didx.at[j0 + b]], ssem[b],
                         add=True)
      for b in range(NBUF):
        wait_scatter(b)
        nxt = j0 + NBUF + b
        gather(jnp.minimum(nxt, FEAT_CHUNKS - 1), b)
      return carry

    lax.fori_loop(0, FEAT_CHUNKS // NBUF, body, 0)
    for b in range(NBUF):
      wait_gather(b)
    plsc.subcore_barrier()

    def stage_out(k, b):
      pltpu.async_copy(acc_sh.at[pl.ds(row0 + k * CHUNK, CHUNK)], rows.at[b],
                       gsem[b])

    def flush_out(k, b):
      pltpu.make_async_copy(zrows_hbm, rows.at[b], gsem[b]).wait()
      pltpu.async_copy(rows.at[b], agg_hbm.at[cid, pl.ds(row0 + k * CHUNK, CHUNK)],
                       ssem[b])

    for k in range(NBUF):
      stage_out(k, k)
    for k in range(NBUF):
      flush_out(k, k)
    for k in range(NBUF, TILE_ROUNDS):
      b = k - NBUF
      pltpu.make_async_copy(zrows_hbm, rows.at[b], ssem[b]).wait()
      stage_out(k, b)
      flush_out(k, b)
    for b in range(NBUF):
      pltpu.make_async_copy(zrows_hbm, rows.at[b], ssem[b]).wait()

  return feat_kernel(xcat, src2r, dstr, zrows)


def _sc_counts(dst2r, zcnt, ones_chunk):
  @functools.partial(
      pl.kernel,
      mesh=plsc.VectorSubcoreMesh(**_SC_MESH),
      compiler_params=pltpu.CompilerParams(use_tc_tiling_on_sc=False),
      out_type=jax.ShapeDtypeStruct((NUM_CORES, N_PAD, CNT_W), jnp.float32),
      scratch_types=[
          pltpu.VMEM((CNT_CHUNKS, CHUNK), jnp.int32),
          pltpu.VMEM((CHUNK, CNT_W), jnp.float32),
          pltpu.VMEM_SHARED((N_PAD, CNT_W), jnp.float32),
          pltpu.SemaphoreType.DMA,
      ],
  )
  def cnt_kernel(dst2r_hbm, zcnt_hbm, ones_hbm, cntp_hbm, didx, ones_v,
                 cnt_sh, sem):
    cid = lax.axis_index("c")
    sid = lax.axis_index("s")
    row0 = sid * ROWS_PER_TILE

    pltpu.sync_copy(dst2r_hbm.at[cid, sid], didx)
    pltpu.sync_copy(zcnt_hbm, ones_v)
    for k in range(TILE_ROUNDS):
      pltpu.sync_copy(ones_v, cnt_sh.at[pl.ds(row0 + k * CHUNK, CHUNK)])
    pltpu.sync_copy(ones_hbm, ones_v)
    plsc.subcore_barrier()

    def body(t, carry):
      j0 = CNT_BATCH * t
      for b in range(CNT_BATCH):
        pltpu.async_copy(ones_v, cnt_sh.at[didx.at[j0 + b]], sem, add=True)
      for b in range(CNT_BATCH):
        pltpu.make_async_copy(zcnt_hbm, ones_v, sem).wait()
      return carry

    lax.fori_loop(0, CNT_CHUNKS // CNT_BATCH, body, 0)
    plsc.subcore_barrier()

    for k in range(TILE_ROUNDS):
      pltpu.sync_copy(cnt_sh.at[pl.ds(row0 + k * CHUNK, CHUNK)], ones_v)
      pltpu.sync_copy(ones_v, cntp_hbm.at[cid, pl.ds(row0 + k * CHUNK, CHUNK)])

  return cnt_kernel(dst2r, zcnt, ones_chunk)


ROW_BLK = 1000


def _head_kernel(x_ref, agg_ref, cntp_ref, wl_ref, bl_ref, wr_ref,
                 wmlp_ref, bmlp_ref, out_ref):
  cnt = cntp_ref[0, :, 0] + cntp_ref[1, :, 0]
  recip = 1.0 / jnp.maximum(cnt, 1.0)
  mean0 = agg_ref[0] * recip[:, None]
  mean1 = agg_ref[1] * recip[:, None]
  wl = wl_ref[...]
  dn = (((1,), (1,)), ((), ()))
  h = lax.dot_general(mean0, wl[:, :HD], dn, preferred_element_type=jnp.float32)
  h += lax.dot_general(mean1, wl[:, HD:], dn, preferred_element_type=jnp.float32)
  h += lax.dot_general(x_ref[...], wr_ref[...], dn,
                       preferred_element_type=jnp.float32)
  h = jnp.maximum(h + bl_ref[...], 0.0)
  out_ref[...] = lax.dot_general(
      h, wmlp_ref[...], dn, preferred_element_type=jnp.float32) + bmlp_ref[...]


def _tc_head(x, agg, cntp, W_l, b_l, W_r, W_mlp, b_mlp):
  grid = (N // ROW_BLK,)
  return pl.pallas_call(
      _head_kernel,
      grid=grid,
      in_specs=[
          pl.BlockSpec((ROW_BLK, D), lambda i: (i, 0)),
          pl.BlockSpec((NUM_CORES, ROW_BLK, HD), lambda i: (0, i, 0)),
          pl.BlockSpec((NUM_CORES, ROW_BLK, CNT_W), lambda i: (0, i, 0)),
          pl.BlockSpec((H, D), lambda i: (0, 0)),
          pl.BlockSpec((1, H), lambda i: (0, 0)),
          pl.BlockSpec((H, D), lambda i: (0, 0)),
          pl.BlockSpec((C, H), lambda i: (0, 0)),
          pl.BlockSpec((1, C), lambda i: (0, 0)),
      ],
      out_specs=pl.BlockSpec((ROW_BLK, C), lambda i: (i, 0)),
      out_shape=jax.ShapeDtypeStruct((N, C), jnp.float32),
  )(x, agg, cntp, W_l, b_l.reshape(1, H), W_r, W_mlp, b_mlp.reshape(1, C))


@jax.jit
def kernel(x, edge_index, W_l, b_l, W_r, W_mlp, b_mlp):
  src = edge_index[0]
  dst = edge_index[1]
  xcat = jnp.concatenate([x[:, :HD], x[:, HD:]], axis=0)
  src2r = jnp.concatenate([src, src + N]).reshape(
      NUM_CORES, NUM_SUBCORES, FEAT_CHUNKS, CHUNK)
  dstr = dst.reshape(NUM_SUBCORES, FEAT_CHUNKS, CHUNK)
  dst2r = dst.reshape(NUM_CORES, NUM_SUBCORES, CNT_CHUNKS, CHUNK)
  zrows = jnp.zeros((CHUNK, HD), jnp.float32)
  zcnt = jnp.zeros((CHUNK, CNT_W), jnp.float32)
  ones_chunk = jnp.ones((CHUNK, CNT_W), jnp.float32)
  agg = _sc_feature_sums(xcat, src2r, dstr, zrows)
  cntp = _sc_counts(dst2r, zcnt, ones_chunk)
  return _tc_head(x, agg, cntp, W_l, b_l, W_r, W_mlp, b_mlp)


# speedup vs baseline: 10.8063x; 1.0500x over previous
import functools

import jax
import jax.numpy as jnp
from jax import lax
from jax.experimental import pallas as pl
from jax.experimental.pallas import tpu as pltpu
from jax.experimental.pallas import tpu_sc as plsc

N = 10000
E = 320000
D = 128
H = 128
C = 64
HD = D // 2

NUM_CORES = 2
NUM_SUBCORES = 16
CHUNK = 80
CNT_W = 8
N_PAD = 10240
ROWS_PER_TILE = N_PAD // NUM_SUBCORES
TILE_ROUNDS = ROWS_PER_TILE // CHUNK

FEAT_EDGES_PER_TILE = E // NUM_SUBCORES
FEAT_CHUNKS = FEAT_EDGES_PER_TILE // CHUNK

_SC_MESH = dict(core_axis_name="c", subcore_axis_name="s")
NBUF = 5


def _sc_aggregate(xcat, src2r, dstr, zrows, zcnt, ones_c):
  @functools.partial(
      pl.kernel,
      mesh=plsc.VectorSubcoreMesh(**_SC_MESH),
      compiler_params=pltpu.CompilerParams(use_tc_tiling_on_sc=False),
      out_type=[
          jax.ShapeDtypeStruct((NUM_CORES, N_PAD, HD), jnp.float32),
          jax.ShapeDtypeStruct((N_PAD, CNT_W), jnp.float32),
      ],
      scratch_types=[
          pltpu.VMEM((FEAT_CHUNKS, CHUNK), jnp.int32),
          pltpu.VMEM((FEAT_CHUNKS, CHUNK), jnp.int32),
          pltpu.VMEM((NBUF, CHUNK, HD), jnp.float32),
          pltpu.VMEM((CHUNK, CNT_W), jnp.float32),
          pltpu.VMEM_SHARED((N_PAD, HD), jnp.float32),
          pltpu.VMEM_SHARED((N_PAD, CNT_W), jnp.float32),
      ] + [pltpu.SemaphoreType.DMA] * (2 * NBUF + 1),
  )
  def agg_kernel(xcat_hbm, src2r_hbm, dstr_hbm, zrows_hbm, zcnt_hbm, ones_hbm,
                 agg_hbm, cnt_hbm, sidx, didx, rows, ones_v, acc_sh, cnt_sh,
                 *sems):
    gsem = sems[:NBUF]
    ssem = sems[NBUF:2 * NBUF]
    csem = sems[2 * NBUF]
    cid = lax.axis_index("c")
    sid = lax.axis_index("s")
    row0 = sid * ROWS_PER_TILE

    pltpu.sync_copy(src2r_hbm.at[cid, sid], sidx)
    pltpu.sync_copy(dstr_hbm.at[sid], didx)

    pltpu.sync_copy(zrows_hbm, rows.at[0])
    pltpu.sync_copy(zcnt_hbm, ones_v)
    for k in range(TILE_ROUNDS):
      pltpu.async_copy(rows.at[0], acc_sh.at[pl.ds(row0 + k * CHUNK, CHUNK)],
                       gsem[0])
      pltpu.async_copy(ones_v, cnt_sh.at[pl.ds(row0 + k * CHUNK, CHUNK)],
                       csem)
    for k in range(TILE_ROUNDS):
      pltpu.make_async_copy(zrows_hbm, rows.at[0], gsem[0]).wait()
      pltpu.make_async_copy(zcnt_hbm, ones_v, csem).wait()
    pltpu.sync_copy(ones_hbm, ones_v)
    plsc.subcore_barrier()

    def gather(j, b):
      pltpu.async_copy(xcat_hbm.at[sidx.at[j]], rows.at[b], gsem[b])

    def wait_gather(b):
      pltpu.make_async_copy(xcat_hbm.at[sidx.at[0]], rows.at[b], gsem[b]).wait()

    def wait_scatter(b):
      pltpu.make_async_copy(zrows_hbm, rows.at[b], ssem[b]).wait()

    for b in range(NBUF):
      gather(b, b)

    def body(t, carry):
      j0 = NBUF * t
      for b in range(NBUF):
        wait_gather(b)
        pltpu.async_copy(rows.at[b], acc_sh.at[didx.at[j0 + b]], ssem[b],
                         add=True)

      @pl.when(cid == 0)
      def _count():
        for b in range(NBUF):
          pltpu.async_copy(ones_v, cnt_sh.at[didx.at[j0 + b]], csem, add=True)

      for b in range(NBUF):
        wait_scatter(b)
        nxt = j0 + NBUF + b
        gather(jnp.minimum(nxt, FEAT_CHUNKS - 1), b)

      @pl.when(cid == 0)
      def _drain_counts():
        for b in range(NBUF):
          pltpu.make_async_copy(zcnt_hbm, ones_v, csem).wait()

      return carry

    lax.fori_loop(0, FEAT_CHUNKS // NBUF, body, 0)
    for b in range(NBUF):
      wait_gather(b)
    plsc.subcore_barrier()

    def stage_out(k, b):
      pltpu.async_copy(acc_sh.at[pl.ds(row0 + k * CHUNK, CHUNK)], rows.at[b],
                       gsem[b])

    def flush_out(k, b):
      pltpu.make_async_copy(zrows_hbm, rows.at[b], gsem[b]).wait()
      pltpu.async_copy(rows.at[b], agg_hbm.at[cid, pl.ds(row0 + k * CHUNK, CHUNK)],
                       ssem[b])

    for k in range(NBUF):
      stage_out(k, k)
    for k in range(NBUF):
      flush_out(k, k)
    for k in range(NBUF, TILE_ROUNDS):
      b = k - NBUF
      pltpu.make_async_copy(zrows_hbm, rows.at[b], ssem[b]).wait()
      stage_out(k, b)
      flush_out(k, b)
    for b in range(NBUF):
      pltpu.make_async_copy(zrows_hbm, rows.at[b], ssem[b]).wait()

    @pl.when(cid == 0)
    def _write_counts():
      for k in range(TILE_ROUNDS):
        pltpu.sync_copy(cnt_sh.at[pl.ds(row0 + k * CHUNK, CHUNK)], ones_v)
        pltpu.sync_copy(ones_v, cnt_hbm.at[pl.ds(row0 + k * CHUNK, CHUNK)])

  return agg_kernel(xcat, src2r, dstr, zrows, zcnt, ones_c)


ROW_BLK = 1000


def _head_kernel(x_ref, agg_ref, cnt_ref, wl_ref, bl_ref, wr_ref,
                 wmlp_ref, bmlp_ref, out_ref):
  recip = 1.0 / jnp.maximum(cnt_ref[:, 0], 1.0)
  mean0 = agg_ref[0] * recip[:, None]
  mean1 = agg_ref[1] * recip[:, None]
  wl = wl_ref[...]
  dn = (((1,), (1,)), ((), ()))
  h = lax.dot_general(mean0, wl[:, :HD], dn, preferred_element_type=jnp.float32)
  h += lax.dot_general(mean1, wl[:, HD:], dn, preferred_element_type=jnp.float32)
  h += lax.dot_general(x_ref[...], wr_ref[...], dn,
                       preferred_element_type=jnp.float32)
  h = jnp.maximum(h + bl_ref[...], 0.0)
  out_ref[...] = lax.dot_general(
      h, wmlp_ref[...], dn, preferred_element_type=jnp.float32) + bmlp_ref[...]


def _tc_head(x, agg, cnt, W_l, b_l, W_r, W_mlp, b_mlp):
  grid = (N // ROW_BLK,)
  return pl.pallas_call(
      _head_kernel,
      grid=grid,
      in_specs=[
          pl.BlockSpec((ROW_BLK, D), lambda i: (i, 0)),
          pl.BlockSpec((NUM_CORES, ROW_BLK, HD), lambda i: (0, i, 0)),
          pl.BlockSpec((ROW_BLK, CNT_W), lambda i: (i, 0)),
          pl.BlockSpec((H, D), lambda i: (0, 0)),
          pl.BlockSpec((1, H), lambda i: (0, 0)),
          pl.BlockSpec((H, D), lambda i: (0, 0)),
          pl.BlockSpec((C, H), lambda i: (0, 0)),
          pl.BlockSpec((1, C), lambda i: (0, 0)),
      ],
      out_specs=pl.BlockSpec((ROW_BLK, C), lambda i: (i, 0)),
      out_shape=jax.ShapeDtypeStruct((N, C), jnp.float32),
  )(x, agg, cnt, W_l, b_l.reshape(1, H), W_r, W_mlp, b_mlp.reshape(1, C))


@jax.jit
def kernel(x, edge_index, W_l, b_l, W_r, W_mlp, b_mlp):
  src = edge_index[0]
  dst = edge_index[1]
  xcat = jnp.concatenate([x[:, :HD], x[:, HD:]], axis=0)
  src2r = jnp.concatenate([src, src + N]).reshape(
      NUM_CORES, NUM_SUBCORES, FEAT_CHUNKS, CHUNK)
  dstr = dst.reshape(NUM_SUBCORES, FEAT_CHUNKS, CHUNK)
  zrows = jnp.zeros((CHUNK, HD), jnp.float32)
  zcnt = jnp.zeros((CHUNK, CNT_W), jnp.float32)
  ones_c = jnp.ones((CHUNK, CNT_W), jnp.float32)
  agg, cnt = _sc_aggregate(xcat, src2r, dstr, zrows, zcnt, ones_c)
  return _tc_head(x, agg, cnt, W_l, b_l, W_r, W_mlp, b_mlp)
